# baseline (speedup 1.0000x reference)
import functools

import jax
import jax.numpy as jnp
from jax.experimental import pallas as pl
from jax.experimental.pallas import tpu as pltpu

_CH = 512
_NB = 8


def _bcast_tc(pe_flat, B, L, D):
    cs = _CH * D
    n = -(-L // _CH)
    sizes = [cs] * (n - 1) + [(L - (n - 1) * _CH) * D]
    total = L * D

    def body(pe_hbm, out_hbm, bufs, rsem, wsem):
        def read(i):
            b = i % _NB
            pltpu.make_async_copy(pe_hbm.at[pl.ds(i * cs, sizes[i])],
                                  bufs.at[b, pl.ds(0, sizes[i])],
                                  rsem.at[b]).start()

        def wait_read(i):
            b = i % _NB
            pltpu.make_async_copy(pe_hbm.at[pl.ds(i * cs, sizes[i])],
                                  bufs.at[b, pl.ds(0, sizes[i])],
                                  rsem.at[b]).wait()

        def write(i):
            b = i % _NB
            for k in range(B):
                pltpu.make_async_copy(bufs.at[b, pl.ds(0, sizes[i])],
                                      out_hbm.at[pl.ds(k * total + i * cs, sizes[i])],
                                      wsem.at[b]).start()

        def wait_write(i):
            b = i % _NB
            for k in range(B):
                pltpu.make_async_copy(bufs.at[b, pl.ds(0, sizes[i])],
                                      out_hbm.at[pl.ds(k * total + i * cs, sizes[i])],
                                      wsem.at[b]).wait()

        for i in range(min(_NB, n)):
            read(i)
        for i in range(n):
            wait_read(i)
            write(i)
            nxt = i + _NB
            if nxt < n:
                wait_write(i)
                read(nxt)
        for i in range(max(0, n - _NB), n):
            wait_write(i)

    return pl.pallas_call(
        body,
        in_specs=[pl.BlockSpec(memory_space=pltpu.HBM)],
        out_specs=pl.BlockSpec(memory_space=pltpu.HBM),
        out_shape=jax.ShapeDtypeStruct((B * total,), jnp.float32),
        scratch_shapes=[
            pltpu.VMEM((_NB, cs), jnp.float32),
            pltpu.SemaphoreType.DMA((_NB,)),
            pltpu.SemaphoreType.DMA((_NB,)),
        ],
    )(pe_flat)


def kernel(x, pe):
    B, S, D = x.shape
    L = 2 * S - 1
    out = _bcast_tc(pe.reshape(-1), B, L, D)
    return out.reshape(B, L, D)


# speedup vs baseline: 4.1391x; 4.1391x over previous
import functools

import jax
import jax.numpy as jnp
from jax.experimental import pallas as pl
from jax.experimental.pallas import tpu as pltpu

_CH = 512
_NB = 8


def _bcast_main(pe2d, B, L, D, n):
    def body(pe_hbm, out_hbm, bufs, rsem, wsem):
        def read(i):
            b = i % _NB
            pltpu.make_async_copy(pe_hbm.at[pl.ds(i * _CH, _CH), :],
                                  bufs.at[b], rsem.at[b]).start()

        def wait_read(i):
            b = i % _NB
            pltpu.make_async_copy(pe_hbm.at[pl.ds(i * _CH, _CH), :],
                                  bufs.at[b], rsem.at[b]).wait()

        def write(i, start=True):
            b = i % _NB
            for k in range(B):
                cp = pltpu.make_async_copy(
                    bufs.at[b],
                    out_hbm.at[k, pl.ds(i * _CH, _CH), :],
                    wsem.at[b])
                cp.start() if start else cp.wait()

        for i in range(min(_NB, n)):
            read(i)
        for i in range(n):
            wait_read(i)
            write(i)
            nxt = i + _NB
            if nxt < n:
                write(i, start=False)
                read(nxt)
        for i in range(max(0, n - _NB), n):
            write(i, start=False)

    return pl.pallas_call(
        body,
        in_specs=[pl.BlockSpec(memory_space=pltpu.HBM)],
        out_specs=pl.BlockSpec(memory_space=pltpu.HBM),
        out_shape=jax.ShapeDtypeStruct((B, L, D), jnp.float32),
        scratch_shapes=[
            pltpu.VMEM((_NB, _CH, D), jnp.float32),
            pltpu.SemaphoreType.DMA((_NB,)),
            pltpu.SemaphoreType.DMA((_NB,)),
        ],
    )(pe2d)


def _bcast_tail(out_main, pe3d, B, L, D, n):
    def body(_, pe_ref, out_ref):
        out_ref[...] = jnp.broadcast_to(pe_ref[...], (B, _CH, D))

    return pl.pallas_call(
        body,
        grid=(1,),
        in_specs=[
            pl.BlockSpec(memory_space=pltpu.HBM),
            pl.BlockSpec((1, _CH, D), lambda i: (0, n, 0)),
        ],
        out_specs=pl.BlockSpec((B, _CH, D), lambda i: (0, n, 0)),
        out_shape=jax.ShapeDtypeStruct((B, L, D), jnp.float32),
        input_output_aliases={0: 0},
    )(out_main, pe3d)


def kernel(x, pe):
    B, S, D = x.shape
    L = 2 * S - 1
    n = L // _CH
    out = _bcast_main(pe[0], B, L, D, n)
    return _bcast_tail(out, pe, B, L, D, n)
